# baseline (speedup 1.0000x reference)
import functools

import jax
import jax.numpy as jnp
from jax import lax
from jax.experimental import pallas as pl
from jax.experimental.pallas import tpu as pltpu
from jax.experimental.pallas import tpu_sc as plsc

F = 26
V = 100000
D = 32
B = 16384

NC, NS = 2, 16
NW = NC * NS
BPW = B // NW
NG = 7
DT = D // 8
TQ = 128 * 8 * 128
WQ = BPW * 8
TW = DT * WQ


@functools.cache
def _build():
    mesh = plsc.VectorSubcoreMesh(
        core_axis_name="c", subcore_axis_name="s", num_cores=NC, num_subcores=NS
    )
    return functools.partial(
        pl.kernel,
        out_type=jax.ShapeDtypeStruct((F, DT, TQ), jnp.float32),
        mesh=mesh,
        scratch_types=[
            pltpu.VMEM((F, BPW), jnp.int32),
            pltpu.VMEM((BPW, D), jnp.float32),
            pltpu.VMEM((BPW, D), jnp.float32),
            pltpu.VMEM((TW,), jnp.float32),
            pltpu.VMEM((TW,), jnp.float32),
            pltpu.SemaphoreType.DMA,
            pltpu.SemaphoreType.DMA,
            pltpu.SemaphoreType.DMA,
            pltpu.SemaphoreType.DMA,
        ],
        compiler_params=pltpu.CompilerParams(
            use_tc_tiling_on_sc=False, needs_layout_passes=False
        ),
    )(_embed_gather)


def _embed_gather(xt_hbm, *args):
    tabs = args[:NG]
    out_hbm = args[NG]
    idx_v, rows0, rows1, t0, t1, g0, g1, w0, w1 = args[NG + 1:]

    wid = lax.axis_index("s") * NC + lax.axis_index("c")
    b0 = wid * BPW

    pltpu.sync_copy(xt_hbm.at[:, pl.ds(b0, BPW)], idx_v)

    def add_off(j, carry):
        f = j // (BPW // 16)
        l = j - f * (BPW // 16)
        sl = (f, pl.ds(l * 16, 16))
        idx_v[sl] = idx_v[sl] * 4 + lax.rem(f, 4)
        return carry

    lax.fori_loop(0, F * (BPW // 16), add_off, 0)

    rows = (rows0, rows1)
    tbuf = (t0, t1)
    gsem = (g0, g1)
    wsem = (w0, w1)

    ii = lax.iota(jnp.int32, 16)

    def transpose(nb):
        r_buf, t_buf = rows[nb], tbuf[nb]

        def body(q, carry):
            p = q * 16
            dt = p >> 12
            bt = (p >> 10) & 3
            ds = (p >> 7) & 7
            bs0 = p & 127
            ridx = bt * 128 + bs0 + ii
            cidx = jnp.broadcast_to(dt * 8 + ds, (16,)).astype(jnp.int32)
            vals = plsc.load_gather(r_buf, [ridx, cidx])
            t_buf[pl.ds(p, 16)] = vals
            return carry

        lax.fori_loop(0, TW // 16, body, 0)

    def fire_gather(f, nb):
        pltpu.async_copy(tabs[f // 4].at[idx_v.at[f]], rows[nb], gsem[nb])

    def wait_gather(f, nb):
        pltpu.make_async_copy(
            tabs[f // 4].at[idx_v.at[f]], rows[nb], gsem[nb]
        ).wait()

    def _write_parts(f, nb):
        for dt in range(DT):
            yield (
                tbuf[nb].at[pl.ds(dt * WQ, WQ)],
                out_hbm.at[f, dt, pl.ds(wid * WQ, WQ)],
            )

    def fire_write(f, nb):
        for src, dst in _write_parts(f, nb):
            pltpu.async_copy(src, dst, wsem[nb])

    def wait_write(f, nb):
        for src, dst in _write_parts(f, nb):
            pltpu.make_async_copy(src, dst, wsem[nb]).wait()

    fire_gather(0, 0)
    for f in range(F):
        nb = f % 2
        if f + 1 < F:
            fire_gather(f + 1, 1 - nb)
        wait_gather(f, nb)
        if f >= 2:
            wait_write(f - 2, nb)
        transpose(nb)
        fire_write(f, nb)
    wait_write(F - 2, (F - 2) % 2)
    wait_write(F - 1, (F - 1) % 2)


def kernel(x_sparse, tables):
    xt = jnp.transpose(x_sparse.astype(jnp.int32))
    tt = jnp.transpose(tables, (0, 2, 1)).reshape(F * D, V)
    groups = []
    for g in range(NG - 1):
        tg = jnp.transpose(tt[g * 128:(g + 1) * 128, :])
        groups.append(tg.reshape(4 * V, D))
    tail = jnp.pad(tt[(NG - 1) * 128:, :], ((0, 64), (0, 0)))
    groups.append(jnp.transpose(tail).reshape(4 * V, D))
    out5 = _build()(xt, *groups)
    out5 = out5.reshape(F, DT, B // 128, 8, 128)
    out = jnp.transpose(out5, (2, 4, 0, 1, 3))
    return out.reshape(B, F, D)
